# baseline (speedup 1.0000x reference)
import functools

import jax
import jax.numpy as jnp
from jax import lax
from jax.experimental import pallas as pl

_NQ = 8


def _rvq_body(x_ref, cb_ref, c2_ref, out_ref):
    r = x_ref[...]
    T = r.shape[0]
    K = cb_ref.shape[1]
    iota_k = lax.broadcasted_iota(jnp.int32, (T, K), 1)
    for q in range(_NQ):
        cb = cb_ref[q]
        dots = lax.dot_general(
            r, cb, (((1,), (1,)), ((), ())),
            preferred_element_type=jnp.float32)
        r2 = jnp.sum(r * r, axis=1, keepdims=True)
        dist = (r2 - 2.0 * dots) + c2_ref[q][None, :]
        m = jnp.min(dist, axis=1, keepdims=True)
        idx = jnp.min(jnp.where(dist == m, iota_k, K), axis=1)
        onehot = (iota_k == idx[:, None]).astype(jnp.float32)
        quant = lax.dot_general(
            onehot, cb, (((1,), (0,)), ((), ())),
            precision=lax.Precision.HIGHEST,
            preferred_element_type=jnp.float32)
        r = r - quant
    out_ref[...] = x_ref[...] - r


@jax.jit
def kernel(x, codebooks):
    B, N, D = x.shape
    NQ, K, _ = codebooks.shape
    rows = B * N
    T = 1152 if rows % 1152 == 0 else rows
    grid = rows // T
    x2 = x.reshape(rows, D)
    c2 = jnp.sum(codebooks ** 2, axis=-1)
    out = pl.pallas_call(
        _rvq_body,
        grid=(grid,),
        in_specs=[
            pl.BlockSpec((T, D), lambda i: (i, 0)),
            pl.BlockSpec((NQ, K, D), lambda i: (0, 0, 0)),
            pl.BlockSpec((NQ, K), lambda i: (0, 0)),
        ],
        out_specs=pl.BlockSpec((T, D), lambda i: (i, 0)),
        out_shape=jax.ShapeDtypeStruct((rows, D), jnp.float32),
    )(x2, codebooks, c2)
    return out.reshape(B, N, D)


# speedup vs baseline: 1.5346x; 1.5346x over previous
import functools

import jax
import jax.numpy as jnp
from jax import lax
from jax.experimental import pallas as pl

_NQ = 8


def _rvq_body(x_ref, cb_ref, c2_ref, out_ref):
    r = x_ref[...]
    T = r.shape[0]
    K = cb_ref.shape[1]
    iota_k = lax.broadcasted_iota(jnp.int32, (T, K), 1)
    for q in range(_NQ):
        cb = cb_ref[q]
        dots = lax.dot_general(
            r, cb, (((1,), (1,)), ((), ())),
            preferred_element_type=jnp.float32)
        r2 = jnp.sum(r * r, axis=1, keepdims=True)
        dist = (r2 - 2.0 * dots) + c2_ref[q][None, :]
        m = jnp.min(dist, axis=1, keepdims=True)
        idx = jnp.min(jnp.where(dist == m, iota_k, K), axis=1)
        onehot = (iota_k == idx[:, None]).astype(jnp.float32)
        b1 = cb.astype(jnp.bfloat16).astype(jnp.float32)
        t = cb - b1
        b2 = t.astype(jnp.bfloat16).astype(jnp.float32)
        b3 = t - b2
        dn = (((1,), (0,)), ((), ()))
        quant = (
            lax.dot_general(onehot, b1, dn, preferred_element_type=jnp.float32)
            + lax.dot_general(onehot, b2, dn, preferred_element_type=jnp.float32)
        ) + lax.dot_general(onehot, b3, dn, preferred_element_type=jnp.float32)
        r = r - quant
    out_ref[...] = x_ref[...] - r


@jax.jit
def kernel(x, codebooks):
    B, N, D = x.shape
    NQ, K, _ = codebooks.shape
    rows = B * N
    T = 1152 if rows % 1152 == 0 else rows
    grid = rows // T
    x2 = x.reshape(rows, D)
    c2 = jnp.sum(codebooks ** 2, axis=-1)
    out = pl.pallas_call(
        _rvq_body,
        grid=(grid,),
        in_specs=[
            pl.BlockSpec((T, D), lambda i: (i, 0)),
            pl.BlockSpec((NQ, K, D), lambda i: (0, 0, 0)),
            pl.BlockSpec((NQ, K), lambda i: (0, 0)),
        ],
        out_specs=pl.BlockSpec((T, D), lambda i: (i, 0)),
        out_shape=jax.ShapeDtypeStruct((rows, D), jnp.float32),
    )(x2, codebooks, c2)
    return out.reshape(B, N, D)


# speedup vs baseline: 1.6439x; 1.0712x over previous
import functools

import jax
import jax.numpy as jnp
from jax import lax
from jax.experimental import pallas as pl
from jax.experimental.pallas import tpu as pltpu
from jax.experimental.pallas import tpu_sc as plsc

_NQ = 8
_NC, _NS = 2, 16
_NW = _NC * _NS


def _argmin_body(r_ref, cb_ref, c2_ref, idx_ref):
    r = r_ref[...]
    T = r.shape[0]
    K = cb_ref.shape[0]
    iota_k = lax.broadcasted_iota(jnp.int32, (T, K), 1)
    dots = lax.dot_general(
        r, cb_ref[...], (((1,), (1,)), ((), ())),
        preferred_element_type=jnp.float32)
    r2 = jnp.sum(r * r, axis=1, keepdims=True)
    dist = (r2 - 2.0 * dots) + c2_ref[0][None, :]
    m = jnp.min(dist, axis=1, keepdims=True)
    idx_ref[0, 0, :] = jnp.min(jnp.where(dist == m, iota_k, K), axis=1)


def _sub_argmin_body(r_ref, q_ref, cb_ref, c2_ref, r_out_ref, idx_ref):
    r = r_ref[...] - q_ref[...]
    r_out_ref[...] = r
    T = r.shape[0]
    K = cb_ref.shape[0]
    iota_k = lax.broadcasted_iota(jnp.int32, (T, K), 1)
    dots = lax.dot_general(
        r, cb_ref[...], (((1,), (1,)), ((), ())),
        preferred_element_type=jnp.float32)
    r2 = jnp.sum(r * r, axis=1, keepdims=True)
    dist = (r2 - 2.0 * dots) + c2_ref[0][None, :]
    m = jnp.min(dist, axis=1, keepdims=True)
    idx_ref[0, 0, :] = jnp.min(jnp.where(dist == m, iota_k, K), axis=1)


def _tc_argmin(r, cb, c2row, T):
    rows, D = r.shape
    K = cb.shape[0]
    G = rows // T
    idx = pl.pallas_call(
        _argmin_body,
        grid=(G,),
        in_specs=[
            pl.BlockSpec((T, D), lambda i: (i, 0)),
            pl.BlockSpec((K, D), lambda i: (0, 0)),
            pl.BlockSpec((1, K), lambda i: (0, 0)),
        ],
        out_specs=pl.BlockSpec((1, 1, T), lambda i: (i, 0, 0)),
        out_shape=jax.ShapeDtypeStruct((G, 1, T), jnp.int32),
    )(r, cb, c2row)
    return idx.reshape(rows)


def _tc_sub_argmin(r, q, cb, c2row, T):
    rows, D = r.shape
    K = cb.shape[0]
    G = rows // T
    r_out, idx = pl.pallas_call(
        _sub_argmin_body,
        grid=(G,),
        in_specs=[
            pl.BlockSpec((T, D), lambda i: (i, 0)),
            pl.BlockSpec((T, D), lambda i: (i, 0)),
            pl.BlockSpec((K, D), lambda i: (0, 0)),
            pl.BlockSpec((1, K), lambda i: (0, 0)),
        ],
        out_specs=[
            pl.BlockSpec((T, D), lambda i: (i, 0)),
            pl.BlockSpec((1, 1, T), lambda i: (i, 0, 0)),
        ],
        out_shape=[
            jax.ShapeDtypeStruct((rows, D), jnp.float32),
            jax.ShapeDtypeStruct((G, 1, T), jnp.int32),
        ],
    )(r, q, cb, c2row)
    return r_out, idx.reshape(rows)


def _make_sc_gather(rows, D, K, nb, bsz):
    mesh = plsc.VectorSubcoreMesh(core_axis_name="c", subcore_axis_name="s")

    @functools.partial(
        pl.kernel,
        mesh=mesh,
        out_type=jax.ShapeDtypeStruct((rows, D), jnp.float32),
        scratch_types=[
            pltpu.VMEM((nb * bsz,), jnp.int32),
            pltpu.VMEM((bsz, D), jnp.float32),
            pltpu.SemaphoreType.DMA,
        ],
    )
    def sc_gather(cb_hbm, idx_hbm, out_hbm, idx_v, rows_v, sem):
        wid = lax.axis_index("s") * _NC + lax.axis_index("c")
        base = wid * nb * bsz
        pltpu.sync_copy(idx_hbm.at[pl.ds(base, nb * bsz)], idx_v)
        for h in range(nb):
            pltpu.async_copy(
                cb_hbm.at[idx_v.at[pl.ds(h * bsz, bsz)]], rows_v, sem).wait()
            pltpu.sync_copy(rows_v, out_hbm.at[pl.ds(base + h * bsz, bsz)])

    return sc_gather


@jax.jit
def kernel(x, codebooks):
    B, N, D = x.shape
    NQ, K, _ = codebooks.shape
    rows = B * N
    T = 1152 if rows % 1152 == 0 else rows
    bsz = 96
    nb = rows // (_NW * bsz)
    x2 = x.reshape(rows, D)
    c2 = jnp.sum(codebooks ** 2, axis=-1)
    sc_gather = _make_sc_gather(rows, D, K, nb, bsz)

    r = x2
    q_rows = None
    for q in range(_NQ):
        if q == 0:
            idx = _tc_argmin(r, codebooks[0], c2[0:1], T)
        else:
            r, idx = _tc_sub_argmin(r, q_rows, codebooks[q], c2[q:q + 1], T)
        q_rows = sc_gather(codebooks[q], idx)
    out = (x2 - r) + q_rows
    return out.reshape(B, N, D)


# speedup vs baseline: 1.6746x; 1.0186x over previous
import functools

import jax
import jax.numpy as jnp
from jax import lax
from jax.experimental import pallas as pl
from jax.experimental.pallas import tpu as pltpu
from jax.experimental.pallas import tpu_sc as plsc

_NQ = 8
_NC, _NS = 2, 16
_NW = _NC * _NS


def _argmin_body(r_ref, cb_ref, c2_ref, idx_ref):
    r = r_ref[...]
    T = r.shape[0]
    K = cb_ref.shape[0]
    iota_k = lax.broadcasted_iota(jnp.int32, (T, K), 1)
    dots = lax.dot_general(
        r, cb_ref[...], (((1,), (1,)), ((), ())),
        preferred_element_type=jnp.float32)
    r2 = jnp.sum(r * r, axis=1, keepdims=True)
    dist = (r2 - 2.0 * dots) + c2_ref[0][None, :]
    m = jnp.min(dist, axis=1, keepdims=True)
    idx_ref[0, 0, :] = jnp.min(jnp.where(dist == m, iota_k, K), axis=1)


def _sub_argmin_body(r_ref, q_ref, cb_ref, c2_ref, r_out_ref, idx_ref):
    r = r_ref[...] - q_ref[...]
    r_out_ref[...] = r
    T = r.shape[0]
    K = cb_ref.shape[0]
    iota_k = lax.broadcasted_iota(jnp.int32, (T, K), 1)
    dots = lax.dot_general(
        r, cb_ref[...], (((1,), (1,)), ((), ())),
        preferred_element_type=jnp.float32)
    r2 = jnp.sum(r * r, axis=1, keepdims=True)
    dist = (r2 - 2.0 * dots) + c2_ref[0][None, :]
    m = jnp.min(dist, axis=1, keepdims=True)
    idx_ref[0, 0, :] = jnp.min(jnp.where(dist == m, iota_k, K), axis=1)


def _tc_argmin(r, cb, c2row, T):
    rows, D = r.shape
    K = cb.shape[0]
    G = rows // T
    idx = pl.pallas_call(
        _argmin_body,
        grid=(G,),
        in_specs=[
            pl.BlockSpec((T, D), lambda i: (i, 0)),
            pl.BlockSpec((K, D), lambda i: (0, 0)),
            pl.BlockSpec((1, K), lambda i: (0, 0)),
        ],
        out_specs=pl.BlockSpec((1, 1, T), lambda i: (i, 0, 0)),
        out_shape=jax.ShapeDtypeStruct((G, 1, T), jnp.int32),
    )(r, cb, c2row)
    return idx.reshape(rows)


def _tc_sub_argmin(r, q, cb, c2row, T):
    rows, D = r.shape
    K = cb.shape[0]
    G = rows // T
    r_out, idx = pl.pallas_call(
        _sub_argmin_body,
        grid=(G,),
        in_specs=[
            pl.BlockSpec((T, D), lambda i: (i, 0)),
            pl.BlockSpec((T, D), lambda i: (i, 0)),
            pl.BlockSpec((K, D), lambda i: (0, 0)),
            pl.BlockSpec((1, K), lambda i: (0, 0)),
        ],
        out_specs=[
            pl.BlockSpec((T, D), lambda i: (i, 0)),
            pl.BlockSpec((1, 1, T), lambda i: (i, 0, 0)),
        ],
        out_shape=[
            jax.ShapeDtypeStruct((rows, D), jnp.float32),
            jax.ShapeDtypeStruct((G, 1, T), jnp.int32),
        ],
    )(r, q, cb, c2row)
    return r_out, idx.reshape(rows)


def _make_sc_gather(rows, D, K, nb, bsz):
    mesh = plsc.VectorSubcoreMesh(core_axis_name="c", subcore_axis_name="s")

    @functools.partial(
        pl.kernel,
        mesh=mesh,
        out_type=jax.ShapeDtypeStruct((rows, D), jnp.float32),
        scratch_types=(
            [pltpu.VMEM((nb * bsz,), jnp.int32)]
            + [pltpu.VMEM((bsz, D), jnp.float32) for _ in range(nb)]
            + [pltpu.SemaphoreType.DMA for _ in range(nb)]
            + [pltpu.SemaphoreType.DMA]
        ),
    )
    def sc_gather(cb_hbm, idx_hbm, out_hbm, idx_v, *bufs_sems):
        bufs = bufs_sems[:nb]
        gsems = bufs_sems[nb:2 * nb]
        ssem = bufs_sems[2 * nb]
        wid = lax.axis_index("s") * _NC + lax.axis_index("c")
        base = wid * nb * bsz
        pltpu.sync_copy(idx_hbm.at[pl.ds(base, nb * bsz)], idx_v)
        gathers = [
            pltpu.async_copy(
                cb_hbm.at[idx_v.at[pl.ds(h * bsz, bsz)]], bufs[h], gsems[h])
            for h in range(nb)
        ]
        stores = []
        for h in range(nb):
            gathers[h].wait()
            stores.append(
                pltpu.async_copy(
                    bufs[h], out_hbm.at[pl.ds(base + h * bsz, bsz)], ssem))
        for st in stores:
            st.wait()

    return sc_gather


@jax.jit
def kernel(x, codebooks):
    B, N, D = x.shape
    NQ, K, _ = codebooks.shape
    rows = B * N
    T = 1152 if rows % 1152 == 0 else rows
    bsz = 96
    nb = rows // (_NW * bsz)
    x2 = x.reshape(rows, D)
    c2 = jnp.sum(codebooks ** 2, axis=-1)
    sc_gather = _make_sc_gather(rows, D, K, nb, bsz)

    r = x2
    q_rows = None
    for q in range(_NQ):
        if q == 0:
            idx = _tc_argmin(r, codebooks[0], c2[0:1], T)
        else:
            r, idx = _tc_sub_argmin(r, q_rows, codebooks[q], c2[q:q + 1], T)
        q_rows = sc_gather(codebooks[q], idx)
    out = (x2 - r) + q_rows
    return out.reshape(B, N, D)


# speedup vs baseline: 1.9067x; 1.1386x over previous
import functools

import jax
import jax.numpy as jnp
from jax import lax
from jax.experimental import pallas as pl
from jax.experimental.pallas import tpu as pltpu
from jax.experimental.pallas import tpu_sc as plsc

_NQ = 8
_NC, _NS = 2, 16
_NW = _NC * _NS


def _argmin_body(r_ref, cb_ref, c2_ref, idx_ref):
    r = r_ref[...]
    T = r.shape[0]
    K = cb_ref.shape[0]
    iota_k = lax.broadcasted_iota(jnp.int32, (T, K), 1)
    dots = lax.dot_general(
        r, cb_ref[...], (((1,), (1,)), ((), ())),
        preferred_element_type=jnp.float32)
    r2 = jnp.sum(r * r, axis=1, keepdims=True)
    dist = (r2 - 2.0 * dots) + c2_ref[0][None, :]
    m = jnp.min(dist, axis=1, keepdims=True)
    idx_ref[0, 0, :] = jnp.min(jnp.where(dist == m, iota_k, K), axis=1)


def _sub_argmin_body(r_ref, q_ref, cb_ref, c2_ref, r_out_ref, idx_ref):
    r = r_ref[...] - q_ref[...]
    r_out_ref[...] = r
    T = r.shape[0]
    K = cb_ref.shape[0]
    iota_k = lax.broadcasted_iota(jnp.int32, (T, K), 1)
    dots = lax.dot_general(
        r, cb_ref[...], (((1,), (1,)), ((), ())),
        preferred_element_type=jnp.float32)
    r2 = jnp.sum(r * r, axis=1, keepdims=True)
    dist = (r2 - 2.0 * dots) + c2_ref[0][None, :]
    m = jnp.min(dist, axis=1, keepdims=True)
    idx_ref[0, 0, :] = jnp.min(jnp.where(dist == m, iota_k, K), axis=1)


def _tc_argmin(r, cb, c2row, T):
    rows, D = r.shape
    K = cb.shape[0]
    G = rows // T
    idx = pl.pallas_call(
        _argmin_body,
        grid=(G,),
        in_specs=[
            pl.BlockSpec((T, D), lambda i: (i, 0)),
            pl.BlockSpec((K, D), lambda i: (0, 0)),
            pl.BlockSpec((1, K), lambda i: (0, 0)),
        ],
        out_specs=pl.BlockSpec((1, 1, T), lambda i: (i, 0, 0)),
        out_shape=jax.ShapeDtypeStruct((G, 1, T), jnp.int32),
    )(r, cb, c2row)
    return idx.reshape(rows)


def _tc_sub_argmin(r, q, cb, c2row, T):
    rows, D = r.shape
    K = cb.shape[0]
    G = rows // T
    r_out, idx = pl.pallas_call(
        _sub_argmin_body,
        grid=(G,),
        in_specs=[
            pl.BlockSpec((T, D), lambda i: (i, 0)),
            pl.BlockSpec((T, D), lambda i: (i, 0)),
            pl.BlockSpec((K, D), lambda i: (0, 0)),
            pl.BlockSpec((1, K), lambda i: (0, 0)),
        ],
        out_specs=[
            pl.BlockSpec((T, D), lambda i: (i, 0)),
            pl.BlockSpec((1, 1, T), lambda i: (i, 0, 0)),
        ],
        out_shape=[
            jax.ShapeDtypeStruct((rows, D), jnp.float32),
            jax.ShapeDtypeStruct((G, 1, T), jnp.int32),
        ],
    )(r, q, cb, c2row)
    return r_out, idx.reshape(rows)


def _make_sc_gather(rows, D, K, nb, bsz):
    mesh = plsc.VectorSubcoreMesh(core_axis_name="c", subcore_axis_name="s")

    @functools.partial(
        pl.kernel,
        mesh=mesh,
        out_type=jax.ShapeDtypeStruct((rows, D), jnp.float32),
        scratch_types=(
            [pltpu.VMEM((nb * bsz,), jnp.int32)]
            + [pltpu.VMEM((bsz, D), jnp.float32) for _ in range(nb)]
            + [pltpu.SemaphoreType.DMA for _ in range(nb)]
            + [pltpu.SemaphoreType.DMA]
        ),
    )
    def sc_gather(cb_hbm, idx_hbm, out_hbm, idx_v, *bufs_sems):
        bufs = bufs_sems[:nb]
        gsems = bufs_sems[nb:2 * nb]
        ssem = bufs_sems[2 * nb]
        wid = lax.axis_index("s") * _NC + lax.axis_index("c")
        base = wid * nb * bsz
        pltpu.sync_copy(idx_hbm.at[pl.ds(base, nb * bsz)], idx_v)
        gathers = [
            pltpu.async_copy(
                cb_hbm.at[idx_v.at[pl.ds(h * bsz, bsz)]], bufs[h], gsems[h])
            for h in range(nb)
        ]
        stores = []
        for h in range(nb):
            gathers[h].wait()
            stores.append(
                pltpu.async_copy(
                    bufs[h], out_hbm.at[pl.ds(base + h * bsz, bsz)], ssem))
        for st in stores:
            st.wait()

    return sc_gather


@jax.jit
def kernel(x, codebooks):
    B, N, D = x.shape
    NQ, K, _ = codebooks.shape
    rows = B * N
    nch = 2 if rows % (2 * _NW * 8) == 0 else 1
    crows = rows // nch
    T = 1152 if crows % 1152 == 0 else crows
    per_w = crows // _NW
    bsz = per_w
    while bsz > 128:
        bsz //= 2
    nb = per_w // bsz
    x2 = x.reshape(rows, D)
    c2 = jnp.sum(codebooks ** 2, axis=-1)
    sc_gather = _make_sc_gather(crows, D, K, nb, bsz)

    xc = [x2[ch * crows:(ch + 1) * crows] for ch in range(nch)]
    r = list(xc)
    q_rows = [None] * nch
    for q in range(_NQ):
        for ch in range(nch):
            if q == 0:
                idx = _tc_argmin(r[ch], codebooks[0], c2[0:1], T)
            else:
                r[ch], idx = _tc_sub_argmin(
                    r[ch], q_rows[ch], codebooks[q], c2[q:q + 1], T)
            q_rows[ch] = sc_gather(codebooks[q], idx)
    out = jnp.concatenate(
        [(xc[ch] - r[ch]) + q_rows[ch] for ch in range(nch)], axis=0)
    return out.reshape(B, N, D)


# speedup vs baseline: 1.9735x; 1.0350x over previous
import functools

import jax
import jax.numpy as jnp
from jax import lax
from jax.experimental import pallas as pl
from jax.experimental.pallas import tpu as pltpu
from jax.experimental.pallas import tpu_sc as plsc

_NQ = 8
_NC, _NS = 2, 16
_NW = _NC * _NS


def _argmin_first_body(x_ref, cb_ref, c2_ref, idx_ref, *, layer):
    r = x_ref[...]
    T = r.shape[0]
    K = cb_ref.shape[1]
    iota_g = lax.broadcasted_iota(jnp.int32, (T, K), 1) + layer * K
    dots = lax.dot_general(
        r, cb_ref[0], (((1,), (1,)), ((), ())),
        preferred_element_type=jnp.float32)
    r2 = jnp.sum(r * r, axis=1, keepdims=True)
    dist = (r2 - 2.0 * dots) + c2_ref[0, 0][None, :]
    m = jnp.min(dist, axis=1, keepdims=True)
    idx_ref[...] = jnp.min(jnp.where(dist == m, iota_g, (layer + 1) * K),
                           axis=1)


def _sub_argmin_body(r_ref, q_ref, cb_ref, c2_ref, r_out_ref, idx_ref, *,
                     layer):
    r = r_ref[...] - q_ref[...]
    r_out_ref[...] = r
    T = r.shape[0]
    K = cb_ref.shape[1]
    iota_g = lax.broadcasted_iota(jnp.int32, (T, K), 1) + layer * K
    dots = lax.dot_general(
        r, cb_ref[0], (((1,), (1,)), ((), ())),
        preferred_element_type=jnp.float32)
    r2 = jnp.sum(r * r, axis=1, keepdims=True)
    dist = (r2 - 2.0 * dots) + c2_ref[0, 0][None, :]
    m = jnp.min(dist, axis=1, keepdims=True)
    idx_ref[...] = jnp.min(jnp.where(dist == m, iota_g, (layer + 1) * K),
                           axis=1)


def _sub_argmin_last_body(r_ref, q_ref, x_ref, cb_ref, c2_ref, s_out_ref,
                          idx_ref, *, layer):
    r = r_ref[...] - q_ref[...]
    s_out_ref[...] = x_ref[...] - r
    T = r.shape[0]
    K = cb_ref.shape[1]
    iota_g = lax.broadcasted_iota(jnp.int32, (T, K), 1) + layer * K
    dots = lax.dot_general(
        r, cb_ref[0], (((1,), (1,)), ((), ())),
        preferred_element_type=jnp.float32)
    r2 = jnp.sum(r * r, axis=1, keepdims=True)
    dist = (r2 - 2.0 * dots) + c2_ref[0, 0][None, :]
    m = jnp.min(dist, axis=1, keepdims=True)
    idx_ref[...] = jnp.min(jnp.where(dist == m, iota_g, (layer + 1) * K),
                           axis=1)


def _tc_argmin_first(x2, cbs, c2, layer, ch, T, crows):
    rows, D = x2.shape
    NQ, K, _ = cbs.shape
    G = crows // T
    return pl.pallas_call(
        functools.partial(_argmin_first_body, layer=layer),
        grid=(G,),
        in_specs=[
            pl.BlockSpec((T, D), lambda i, ch=ch, G=G: (ch * G + i, 0)),
            pl.BlockSpec((1, K, D), lambda i, q=layer: (q, 0, 0)),
            pl.BlockSpec((1, 1, K), lambda i, q=layer: (q, 0, 0)),
        ],
        out_specs=pl.BlockSpec((T,), lambda i: (i,)),
        out_shape=jax.ShapeDtypeStruct((crows,), jnp.int32),
    )(x2, cbs, c2)


def _tc_sub_argmin(r, q_rows, cbs, c2, layer, T, roff=0):
    crows, D = q_rows.shape
    NQ, K, _ = cbs.shape
    G = crows // T
    return pl.pallas_call(
        functools.partial(_sub_argmin_body, layer=layer),
        grid=(G,),
        in_specs=[
            pl.BlockSpec((T, D), lambda i, o=roff: (o + i, 0)),
            pl.BlockSpec((T, D), lambda i: (i, 0)),
            pl.BlockSpec((1, K, D), lambda i, q=layer: (q, 0, 0)),
            pl.BlockSpec((1, 1, K), lambda i, q=layer: (q, 0, 0)),
        ],
        out_specs=[
            pl.BlockSpec((T, D), lambda i: (i, 0)),
            pl.BlockSpec((T,), lambda i: (i,)),
        ],
        out_shape=[
            jax.ShapeDtypeStruct((crows, D), jnp.float32),
            jax.ShapeDtypeStruct((crows,), jnp.int32),
        ],
    )(r, q_rows, cbs, c2)


def _tc_sub_argmin_last(r, q_rows, x2, cbs, c2, layer, ch, T):
    crows, D = r.shape
    NQ, K, _ = cbs.shape
    G = crows // T
    return pl.pallas_call(
        functools.partial(_sub_argmin_last_body, layer=layer),
        grid=(G,),
        in_specs=[
            pl.BlockSpec((T, D), lambda i: (i, 0)),
            pl.BlockSpec((T, D), lambda i: (i, 0)),
            pl.BlockSpec((T, D), lambda i, ch=ch, G=G: (ch * G + i, 0)),
            pl.BlockSpec((1, K, D), lambda i, q=layer: (q, 0, 0)),
            pl.BlockSpec((1, 1, K), lambda i, q=layer: (q, 0, 0)),
        ],
        out_specs=[
            pl.BlockSpec((T, D), lambda i: (i, 0)),
            pl.BlockSpec((T,), lambda i: (i,)),
        ],
        out_shape=[
            jax.ShapeDtypeStruct((crows, D), jnp.float32),
            jax.ShapeDtypeStruct((crows,), jnp.int32),
        ],
    )(r, q_rows, x2, cbs, c2)


def _make_sc_gather(crows, D, nb, bsz):
    mesh = plsc.VectorSubcoreMesh(core_axis_name="c", subcore_axis_name="s")

    @functools.partial(
        pl.kernel,
        mesh=mesh,
        out_type=jax.ShapeDtypeStruct((crows, D), jnp.float32),
        scratch_types=(
            [pltpu.VMEM((nb * bsz,), jnp.int32)]
            + [pltpu.VMEM((bsz, D), jnp.float32) for _ in range(nb)]
            + [pltpu.SemaphoreType.DMA for _ in range(nb)]
            + [pltpu.SemaphoreType.DMA]
        ),
    )
    def sc_gather(cb_hbm, idx_hbm, out_hbm, idx_v, *bufs_sems):
        bufs = bufs_sems[:nb]
        gsems = bufs_sems[nb:2 * nb]
        ssem = bufs_sems[2 * nb]
        wid = lax.axis_index("s") * _NC + lax.axis_index("c")
        base = wid * nb * bsz
        pltpu.sync_copy(idx_hbm.at[pl.ds(base, nb * bsz)], idx_v)
        gathers = [
            pltpu.async_copy(
                cb_hbm.at[idx_v.at[pl.ds(h * bsz, bsz)]], bufs[h], gsems[h])
            for h in range(nb)
        ]
        stores = []
        for h in range(nb):
            gathers[h].wait()
            stores.append(
                pltpu.async_copy(
                    bufs[h], out_hbm.at[pl.ds(base + h * bsz, bsz)], ssem))
        for st in stores:
            st.wait()

    return sc_gather


def _make_sc_final(rows, D, nb, bsz):
    mesh = plsc.VectorSubcoreMesh(core_axis_name="c", subcore_axis_name="s")
    per_w = rows // _NW
    half = _NW // 2

    @functools.partial(
        pl.kernel,
        mesh=mesh,
        out_type=jax.ShapeDtypeStruct((rows, D), jnp.float32),
        scratch_types=[
            pltpu.VMEM((per_w,), jnp.int32),
            pltpu.VMEM((bsz, D), jnp.float32),
            pltpu.VMEM((bsz, D), jnp.float32),
            pltpu.SemaphoreType.DMA,
        ],
    )
    def sc_final(cb_hbm, s0_hbm, s1_hbm, i0_hbm, i1_hbm, out_hbm,
                 idx_v, s_v, g_v, sem):
        wid = lax.axis_index("s") * _NC + lax.axis_index("c")
        lbase = (wid % half) * per_w
        base = wid * per_w

        @pl.when(wid < half)
        def _():
            pltpu.sync_copy(i0_hbm.at[pl.ds(lbase, per_w)], idx_v)

        @pl.when(wid >= half)
        def _():
            pltpu.sync_copy(i1_hbm.at[pl.ds(lbase, per_w)], idx_v)

        for h in range(nb):
            @pl.when(wid < half)
            def _(h=h):
                pltpu.sync_copy(s0_hbm.at[pl.ds(lbase + h * bsz, bsz)], s_v)

            @pl.when(wid >= half)
            def _(h=h):
                pltpu.sync_copy(s1_hbm.at[pl.ds(lbase + h * bsz, bsz)], s_v)

            pltpu.async_copy(
                cb_hbm.at[idx_v.at[pl.ds(h * bsz, bsz)]], g_v, sem).wait()

            def _row(row, _):
                for c in range(0, D, 16):
                    g_v[row, pl.ds(c, 16)] = (
                        s_v[row, pl.ds(c, 16)] + g_v[row, pl.ds(c, 16)])
                return _
            lax.fori_loop(0, bsz, _row, None)
            pltpu.sync_copy(g_v, out_hbm.at[pl.ds(base + h * bsz, bsz)])

    return sc_final


@jax.jit
def kernel(x, codebooks):
    B, N, D = x.shape
    NQ, K, _ = codebooks.shape
    rows = B * N
    nch = 2
    crows = rows // nch
    T = 512 if crows % 512 == 0 else crows
    per_w = crows // _NW
    bsz = per_w
    while bsz > 128:
        bsz //= 2
    nb = per_w // bsz
    x2 = x.reshape(rows, D)
    c2 = jnp.sum(codebooks ** 2, axis=-1).reshape(NQ, 1, K)
    cb_flat = codebooks.reshape(NQ * K, D)
    sc_gather = _make_sc_gather(crows, D, nb, bsz)

    fb = rows // _NW
    fbsz = fb
    while fbsz > 128:
        fbsz //= 2
    sc_final = _make_sc_final(rows, D, fb // fbsz, fbsz)

    r = [None] * nch
    s = [None] * nch
    q_rows = [None] * nch
    idx_last = [None] * nch
    for q in range(_NQ):
        for ch in range(nch):
            if q == 0:
                idx = _tc_argmin_first(x2, codebooks, c2, q, ch, T, crows)
            elif q == _NQ - 1:
                s[ch], idx_last[ch] = _tc_sub_argmin_last(
                    r[ch], q_rows[ch], x2, codebooks, c2, q, ch, T)
                continue
            elif q == 1:
                r[ch], idx = _tc_sub_argmin(
                    x2, q_rows[ch], codebooks, c2, q, T,
                    roff=ch * (crows // T))
            else:
                r[ch], idx = _tc_sub_argmin(
                    r[ch], q_rows[ch], codebooks, c2, q, T)
            q_rows[ch] = sc_gather(cb_flat, idx)
    out = sc_final(cb_flat, s[0], s[1], idx_last[0], idx_last[1])
    return out.reshape(B, N, D)


# speedup vs baseline: 1.9901x; 1.0084x over previous
import functools

import jax
import jax.numpy as jnp
from jax import lax
from jax.experimental import pallas as pl
from jax.experimental.pallas import tpu as pltpu
from jax.experimental.pallas import tpu_sc as plsc

_NQ = 8
_NC, _NS = 2, 16
_NW = _NC * _NS


def _argmin_first_body(x_ref, cb_ref, c2_ref, idx_ref, *, layer):
    r = x_ref[...]
    T = r.shape[0]
    K = cb_ref.shape[1]
    iota_g = lax.broadcasted_iota(jnp.int32, (T, K), 1) + layer * K
    dots = lax.dot_general(
        r, cb_ref[0], (((1,), (1,)), ((), ())),
        preferred_element_type=jnp.float32)
    r2 = jnp.sum(r * r, axis=1, keepdims=True)
    dist = (r2 - 2.0 * dots) + c2_ref[0, 0][None, :]
    m = jnp.min(dist, axis=1, keepdims=True)
    idx_ref[...] = jnp.min(jnp.where(dist == m, iota_g, (layer + 1) * K),
                           axis=1)


def _sub_argmin_body(r_ref, q_ref, cb_ref, c2_ref, r_out_ref, idx_ref, *,
                     layer):
    r = r_ref[...] - q_ref[...]
    r_out_ref[...] = r
    T = r.shape[0]
    K = cb_ref.shape[1]
    iota_g = lax.broadcasted_iota(jnp.int32, (T, K), 1) + layer * K
    dots = lax.dot_general(
        r, cb_ref[0], (((1,), (1,)), ((), ())),
        preferred_element_type=jnp.float32)
    r2 = jnp.sum(r * r, axis=1, keepdims=True)
    dist = (r2 - 2.0 * dots) + c2_ref[0, 0][None, :]
    m = jnp.min(dist, axis=1, keepdims=True)
    idx_ref[...] = jnp.min(jnp.where(dist == m, iota_g, (layer + 1) * K),
                           axis=1)


def _sub_argmin_last_body(r_ref, q_ref, x_ref, cb_ref, c2_ref, s_out_ref,
                          idx_ref, *, layer):
    r = r_ref[...] - q_ref[...]
    s_out_ref[...] = x_ref[...] - r
    T = r.shape[0]
    K = cb_ref.shape[1]
    iota_g = lax.broadcasted_iota(jnp.int32, (T, K), 1) + layer * K
    dots = lax.dot_general(
        r, cb_ref[0], (((1,), (1,)), ((), ())),
        preferred_element_type=jnp.float32)
    r2 = jnp.sum(r * r, axis=1, keepdims=True)
    dist = (r2 - 2.0 * dots) + c2_ref[0, 0][None, :]
    m = jnp.min(dist, axis=1, keepdims=True)
    idx_ref[...] = jnp.min(jnp.where(dist == m, iota_g, (layer + 1) * K),
                           axis=1)


def _tc_argmin_first(x2, cbs, c2, layer, ch, T, crows):
    rows, D = x2.shape
    NQ, K, _ = cbs.shape
    G = crows // T
    return pl.pallas_call(
        functools.partial(_argmin_first_body, layer=layer),
        grid=(G,),
        in_specs=[
            pl.BlockSpec((T, D), lambda i, ch=ch, G=G: (ch * G + i, 0)),
            pl.BlockSpec((1, K, D), lambda i, q=layer: (q, 0, 0)),
            pl.BlockSpec((1, 1, K), lambda i, q=layer: (q, 0, 0)),
        ],
        out_specs=pl.BlockSpec((T,), lambda i: (i,)),
        out_shape=jax.ShapeDtypeStruct((crows,), jnp.int32),
    )(x2, cbs, c2)


def _tc_sub_argmin(r, q_rows, cbs, c2, layer, T, roff=0):
    crows, D = q_rows.shape
    NQ, K, _ = cbs.shape
    G = crows // T
    return pl.pallas_call(
        functools.partial(_sub_argmin_body, layer=layer),
        grid=(G,),
        in_specs=[
            pl.BlockSpec((T, D), lambda i, o=roff: (o + i, 0)),
            pl.BlockSpec((T, D), lambda i: (i, 0)),
            pl.BlockSpec((1, K, D), lambda i, q=layer: (q, 0, 0)),
            pl.BlockSpec((1, 1, K), lambda i, q=layer: (q, 0, 0)),
        ],
        out_specs=[
            pl.BlockSpec((T, D), lambda i: (i, 0)),
            pl.BlockSpec((T,), lambda i: (i,)),
        ],
        out_shape=[
            jax.ShapeDtypeStruct((crows, D), jnp.float32),
            jax.ShapeDtypeStruct((crows,), jnp.int32),
        ],
    )(r, q_rows, cbs, c2)


def _tc_sub_argmin_last(r, q_rows, x2, cbs, c2, layer, ch, T):
    crows, D = r.shape
    NQ, K, _ = cbs.shape
    G = crows // T
    return pl.pallas_call(
        functools.partial(_sub_argmin_last_body, layer=layer),
        grid=(G,),
        in_specs=[
            pl.BlockSpec((T, D), lambda i: (i, 0)),
            pl.BlockSpec((T, D), lambda i: (i, 0)),
            pl.BlockSpec((T, D), lambda i, ch=ch, G=G: (ch * G + i, 0)),
            pl.BlockSpec((1, K, D), lambda i, q=layer: (q, 0, 0)),
            pl.BlockSpec((1, 1, K), lambda i, q=layer: (q, 0, 0)),
        ],
        out_specs=[
            pl.BlockSpec((T, D), lambda i: (i, 0)),
            pl.BlockSpec((T,), lambda i: (i,)),
        ],
        out_shape=[
            jax.ShapeDtypeStruct((crows, D), jnp.float32),
            jax.ShapeDtypeStruct((crows,), jnp.int32),
        ],
    )(r, q_rows, x2, cbs, c2)


def _make_sc_gather(crows, D, nb, bsz):
    mesh = plsc.VectorSubcoreMesh(core_axis_name="c", subcore_axis_name="s")

    @functools.partial(
        pl.kernel,
        mesh=mesh,
        out_type=jax.ShapeDtypeStruct((crows, D), jnp.float32),
        scratch_types=(
            [pltpu.VMEM((nb * bsz,), jnp.int32)]
            + [pltpu.VMEM((bsz, D), jnp.float32) for _ in range(nb)]
            + [pltpu.SemaphoreType.DMA for _ in range(nb)]
            + [pltpu.SemaphoreType.DMA]
        ),
    )
    def sc_gather(cb_hbm, idx_hbm, out_hbm, idx_v, *bufs_sems):
        bufs = bufs_sems[:nb]
        gsems = bufs_sems[nb:2 * nb]
        ssem = bufs_sems[2 * nb]
        wid = lax.axis_index("s") * _NC + lax.axis_index("c")
        base = wid * nb * bsz
        pltpu.sync_copy(idx_hbm.at[pl.ds(base, nb * bsz)], idx_v)
        gathers = [
            pltpu.async_copy(
                cb_hbm.at[idx_v.at[pl.ds(h * bsz, bsz)]], bufs[h], gsems[h])
            for h in range(nb)
        ]
        stores = []
        for h in range(nb):
            gathers[h].wait()
            stores.append(
                pltpu.async_copy(
                    bufs[h], out_hbm.at[pl.ds(base + h * bsz, bsz)], ssem))
        for st in stores:
            st.wait()

    return sc_gather


def _make_sc_final(rows, D, nb, bsz):
    mesh = plsc.VectorSubcoreMesh(core_axis_name="c", subcore_axis_name="s")
    per_w = rows // _NW
    half = _NW // 2

    @functools.partial(
        pl.kernel,
        mesh=mesh,
        out_type=jax.ShapeDtypeStruct((rows, D), jnp.float32),
        scratch_types=(
            [pltpu.VMEM((per_w,), jnp.int32)]
            + [pltpu.VMEM((bsz, D), jnp.float32) for _ in range(2)]
            + [pltpu.VMEM((bsz, D), jnp.float32) for _ in range(2)]
            + [pltpu.SemaphoreType.DMA for _ in range(5)]
        ),
    )
    def sc_final(cb_hbm, s0_hbm, s1_hbm, i0_hbm, i1_hbm, out_hbm,
                 idx_v, *bufs_sems):
        sv = bufs_sems[0:2]
        gv = bufs_sems[2:4]
        ssem, gsem0, gsem1, stsem0, stsem1 = bufs_sems[4:9]
        gsems = (gsem0, gsem1)
        stsems = (stsem0, stsem1)
        wid = lax.axis_index("s") * _NC + lax.axis_index("c")
        lbase = (wid % half) * per_w
        base = wid * per_w

        @pl.when(wid < half)
        def _():
            pltpu.sync_copy(i0_hbm.at[pl.ds(lbase, per_w)], idx_v)

        @pl.when(wid >= half)
        def _():
            pltpu.sync_copy(i1_hbm.at[pl.ds(lbase, per_w)], idx_v)

        def load_s(h):
            b = sv[h % 2]

            @pl.when(wid < half)
            def _():
                pltpu.sync_copy(s0_hbm.at[pl.ds(lbase + h * bsz, bsz)], b)

            @pl.when(wid >= half)
            def _():
                pltpu.sync_copy(s1_hbm.at[pl.ds(lbase + h * bsz, bsz)], b)

        def fire_gather(h):
            return pltpu.async_copy(
                cb_hbm.at[idx_v.at[pl.ds(h * bsz, bsz)]], gv[h % 2],
                gsems[h % 2])

        gh = fire_gather(0)
        load_s(0)
        stores = {}
        for h in range(nb):
            gh.wait()
            def _row(row, _, h=h):
                g, s = gv[h % 2], sv[h % 2]
                for c in range(0, D, 16):
                    g[row, pl.ds(c, 16)] = (
                        s[row, pl.ds(c, 16)] + g[row, pl.ds(c, 16)])
                return _
            lax.fori_loop(0, bsz, _row, None)
            stores[h] = pltpu.async_copy(
                gv[h % 2], out_hbm.at[pl.ds(base + h * bsz, bsz)],
                stsems[h % 2])
            if h + 1 < nb:
                if h - 1 >= 0:
                    stores[h - 1].wait()
                gh = fire_gather(h + 1)
                load_s(h + 1)
        for h in (nb - 2, nb - 1):
            if h >= 0 and h in stores and stores[h] is not None:
                stores[h].wait()

    return sc_final


@jax.jit
def kernel(x, codebooks):
    B, N, D = x.shape
    NQ, K, _ = codebooks.shape
    rows = B * N
    nch = 2
    crows = rows // nch
    T = 512 if crows % 512 == 0 else crows
    per_w = crows // _NW
    bsz = per_w
    while bsz > 128:
        bsz //= 2
    nb = per_w // bsz
    x2 = x.reshape(rows, D)
    c2 = jnp.sum(codebooks ** 2, axis=-1).reshape(NQ, 1, K)
    cb_flat = codebooks.reshape(NQ * K, D)
    sc_gather = _make_sc_gather(crows, D, nb, bsz)

    fb = rows // _NW
    fbsz = fb
    while fbsz > 128:
        fbsz //= 2
    sc_final = _make_sc_final(rows, D, fb // fbsz, fbsz)

    r = [None] * nch
    s = [None] * nch
    q_rows = [None] * nch
    idx_last = [None] * nch
    for q in range(_NQ):
        for ch in range(nch):
            if q == 0:
                idx = _tc_argmin_first(x2, codebooks, c2, q, ch, T, crows)
            elif q == _NQ - 1:
                s[ch], idx_last[ch] = _tc_sub_argmin_last(
                    r[ch], q_rows[ch], x2, codebooks, c2, q, ch, T)
                continue
            elif q == 1:
                r[ch], idx = _tc_sub_argmin(
                    x2, q_rows[ch], codebooks, c2, q, T,
                    roff=ch * (crows // T))
            else:
                r[ch], idx = _tc_sub_argmin(
                    r[ch], q_rows[ch], codebooks, c2, q, T)
            q_rows[ch] = sc_gather(cb_flat, idx)
    out = sc_final(cb_flat, s[0], s[1], idx_last[0], idx_last[1])
    return out.reshape(B, N, D)


# speedup vs baseline: 2.0754x; 1.0429x over previous
import functools

import jax
import jax.numpy as jnp
from jax import lax
from jax.experimental import pallas as pl
from jax.experimental.pallas import tpu as pltpu
from jax.experimental.pallas import tpu_sc as plsc

_NQ = 8
_NC, _NS = 2, 16
_NW = _NC * _NS


def _argmin_first_body(x_ref, cb_ref, c2_ref, idx_ref, *, layer):
    r = x_ref[...]
    T = r.shape[0]
    K = cb_ref.shape[1]
    iota_f = (lax.broadcasted_iota(jnp.int32, (T, K), 1)
              .astype(jnp.float32) + float(layer * K))
    dots = lax.dot_general(
        r, cb_ref[0], (((1,), (1,)), ((), ())),
        preferred_element_type=jnp.float32)
    r2 = jnp.sum(r * r, axis=1, keepdims=True)
    dist = (r2 - 2.0 * dots) + c2_ref[0, 0][None, :]
    m = jnp.min(dist, axis=1, keepdims=True)
    key = iota_f + (dist - m) * jnp.float32(1e30)
    idx_ref[...] = jnp.min(key, axis=1).astype(jnp.int32)


def _sub_argmin_body(r_ref, q_ref, cb_ref, c2_ref, r_out_ref, idx_ref, *,
                     layer):
    r = r_ref[...] - q_ref[...]
    r_out_ref[...] = r
    T = r.shape[0]
    K = cb_ref.shape[1]
    iota_f = (lax.broadcasted_iota(jnp.int32, (T, K), 1)
              .astype(jnp.float32) + float(layer * K))
    dots = lax.dot_general(
        r, cb_ref[0], (((1,), (1,)), ((), ())),
        preferred_element_type=jnp.float32)
    r2 = jnp.sum(r * r, axis=1, keepdims=True)
    dist = (r2 - 2.0 * dots) + c2_ref[0, 0][None, :]
    m = jnp.min(dist, axis=1, keepdims=True)
    key = iota_f + (dist - m) * jnp.float32(1e30)
    idx_ref[...] = jnp.min(key, axis=1).astype(jnp.int32)


def _sub_argmin_last_body(r_ref, q_ref, x_ref, cb_ref, c2_ref, s_out_ref,
                          idx_ref, *, layer):
    r = r_ref[...] - q_ref[...]
    s_out_ref[...] = x_ref[...] - r
    T = r.shape[0]
    K = cb_ref.shape[1]
    iota_f = (lax.broadcasted_iota(jnp.int32, (T, K), 1)
              .astype(jnp.float32) + float(layer * K))
    dots = lax.dot_general(
        r, cb_ref[0], (((1,), (1,)), ((), ())),
        preferred_element_type=jnp.float32)
    r2 = jnp.sum(r * r, axis=1, keepdims=True)
    dist = (r2 - 2.0 * dots) + c2_ref[0, 0][None, :]
    m = jnp.min(dist, axis=1, keepdims=True)
    key = iota_f + (dist - m) * jnp.float32(1e30)
    idx_ref[...] = jnp.min(key, axis=1).astype(jnp.int32)


def _tc_argmin_first(x2, cbs, c2, layer, ch, T, crows):
    rows, D = x2.shape
    NQ, K, _ = cbs.shape
    G = crows // T
    return pl.pallas_call(
        functools.partial(_argmin_first_body, layer=layer),
        grid=(G,),
        in_specs=[
            pl.BlockSpec((T, D), lambda i, ch=ch, G=G: (ch * G + i, 0)),
            pl.BlockSpec((1, K, D), lambda i, q=layer: (q, 0, 0)),
            pl.BlockSpec((1, 1, K), lambda i, q=layer: (q, 0, 0)),
        ],
        out_specs=pl.BlockSpec((T,), lambda i: (i,)),
        out_shape=jax.ShapeDtypeStruct((crows,), jnp.int32),
    )(x2, cbs, c2)


def _tc_sub_argmin(r, q_rows, cbs, c2, layer, T, roff=0):
    crows, D = q_rows.shape
    NQ, K, _ = cbs.shape
    G = crows // T
    return pl.pallas_call(
        functools.partial(_sub_argmin_body, layer=layer),
        grid=(G,),
        in_specs=[
            pl.BlockSpec((T, D), lambda i, o=roff: (o + i, 0)),
            pl.BlockSpec((T, D), lambda i: (i, 0)),
            pl.BlockSpec((1, K, D), lambda i, q=layer: (q, 0, 0)),
            pl.BlockSpec((1, 1, K), lambda i, q=layer: (q, 0, 0)),
        ],
        out_specs=[
            pl.BlockSpec((T, D), lambda i: (i, 0)),
            pl.BlockSpec((T,), lambda i: (i,)),
        ],
        out_shape=[
            jax.ShapeDtypeStruct((crows, D), jnp.float32),
            jax.ShapeDtypeStruct((crows,), jnp.int32),
        ],
    )(r, q_rows, cbs, c2)


def _tc_sub_argmin_last(r, q_rows, x2, cbs, c2, layer, ch, T):
    crows, D = r.shape
    NQ, K, _ = cbs.shape
    G = crows // T
    return pl.pallas_call(
        functools.partial(_sub_argmin_last_body, layer=layer),
        grid=(G,),
        in_specs=[
            pl.BlockSpec((T, D), lambda i: (i, 0)),
            pl.BlockSpec((T, D), lambda i: (i, 0)),
            pl.BlockSpec((T, D), lambda i, ch=ch, G=G: (ch * G + i, 0)),
            pl.BlockSpec((1, K, D), lambda i, q=layer: (q, 0, 0)),
            pl.BlockSpec((1, 1, K), lambda i, q=layer: (q, 0, 0)),
        ],
        out_specs=[
            pl.BlockSpec((T, D), lambda i: (i, 0)),
            pl.BlockSpec((T,), lambda i: (i,)),
        ],
        out_shape=[
            jax.ShapeDtypeStruct((crows, D), jnp.float32),
            jax.ShapeDtypeStruct((crows,), jnp.int32),
        ],
    )(r, q_rows, x2, cbs, c2)


def _make_sc_gather(crows, D, nb, bsz):
    mesh = plsc.VectorSubcoreMesh(core_axis_name="c", subcore_axis_name="s")

    @functools.partial(
        pl.kernel,
        mesh=mesh,
        out_type=jax.ShapeDtypeStruct((crows, D), jnp.float32),
        scratch_types=(
            [pltpu.VMEM((nb * bsz,), jnp.int32)]
            + [pltpu.VMEM((bsz, D), jnp.float32) for _ in range(nb)]
            + [pltpu.SemaphoreType.DMA for _ in range(nb)]
            + [pltpu.SemaphoreType.DMA]
        ),
    )
    def sc_gather(cb_hbm, idx_hbm, out_hbm, idx_v, *bufs_sems):
        bufs = bufs_sems[:nb]
        gsems = bufs_sems[nb:2 * nb]
        ssem = bufs_sems[2 * nb]
        wid = lax.axis_index("s") * _NC + lax.axis_index("c")
        base = wid * nb * bsz
        pltpu.sync_copy(idx_hbm.at[pl.ds(base, nb * bsz)], idx_v)
        gathers = [
            pltpu.async_copy(
                cb_hbm.at[idx_v.at[pl.ds(h * bsz, bsz)]], bufs[h], gsems[h])
            for h in range(nb)
        ]
        stores = []
        for h in range(nb):
            gathers[h].wait()
            stores.append(
                pltpu.async_copy(
                    bufs[h], out_hbm.at[pl.ds(base + h * bsz, bsz)], ssem))
        for st in stores:
            st.wait()

    return sc_gather


def _make_sc_final(rows, D, nb, bsz):
    mesh = plsc.VectorSubcoreMesh(core_axis_name="c", subcore_axis_name="s")
    per_w = rows // _NW
    half = _NW // 2

    @functools.partial(
        pl.kernel,
        mesh=mesh,
        out_type=jax.ShapeDtypeStruct((rows, D), jnp.float32),
        scratch_types=(
            [pltpu.VMEM((per_w,), jnp.int32)]
            + [pltpu.VMEM((bsz, D), jnp.float32) for _ in range(2)]
            + [pltpu.VMEM((bsz, D), jnp.float32) for _ in range(2)]
            + [pltpu.SemaphoreType.DMA for _ in range(5)]
        ),
    )
    def sc_final(cb_hbm, s0_hbm, s1_hbm, i0_hbm, i1_hbm, out_hbm,
                 idx_v, *bufs_sems):
        sv = bufs_sems[0:2]
        gv = bufs_sems[2:4]
        ssem, gsem0, gsem1, stsem0, stsem1 = bufs_sems[4:9]
        gsems = (gsem0, gsem1)
        stsems = (stsem0, stsem1)
        wid = lax.axis_index("s") * _NC + lax.axis_index("c")
        lbase = (wid % half) * per_w
        base = wid * per_w

        @pl.when(wid < half)
        def _():
            pltpu.sync_copy(i0_hbm.at[pl.ds(lbase, per_w)], idx_v)

        @pl.when(wid >= half)
        def _():
            pltpu.sync_copy(i1_hbm.at[pl.ds(lbase, per_w)], idx_v)

        def load_s(h):
            b = sv[h % 2]

            @pl.when(wid < half)
            def _():
                pltpu.sync_copy(s0_hbm.at[pl.ds(lbase + h * bsz, bsz)], b)

            @pl.when(wid >= half)
            def _():
                pltpu.sync_copy(s1_hbm.at[pl.ds(lbase + h * bsz, bsz)], b)

        def fire_gather(h):
            return pltpu.async_copy(
                cb_hbm.at[idx_v.at[pl.ds(h * bsz, bsz)]], gv[h % 2],
                gsems[h % 2])

        gh = fire_gather(0)
        load_s(0)
        stores = {}
        for h in range(nb):
            gh.wait()
            def _row(row, _, h=h):
                g, s = gv[h % 2], sv[h % 2]
                for c in range(0, D, 16):
                    g[row, pl.ds(c, 16)] = (
                        s[row, pl.ds(c, 16)] + g[row, pl.ds(c, 16)])
                return _
            lax.fori_loop(0, bsz, _row, None)
            stores[h] = pltpu.async_copy(
                gv[h % 2], out_hbm.at[pl.ds(base + h * bsz, bsz)],
                stsems[h % 2])
            if h + 1 < nb:
                if h - 1 >= 0:
                    stores[h - 1].wait()
                gh = fire_gather(h + 1)
                load_s(h + 1)
        for h in (nb - 2, nb - 1):
            if h >= 0 and h in stores and stores[h] is not None:
                stores[h].wait()

    return sc_final


@jax.jit
def kernel(x, codebooks):
    B, N, D = x.shape
    NQ, K, _ = codebooks.shape
    rows = B * N
    nch = 2
    crows = rows // nch
    T = 512 if crows % 512 == 0 else crows
    per_w = crows // _NW
    bsz = per_w
    while bsz > 128:
        bsz //= 2
    nb = per_w // bsz
    x2 = x.reshape(rows, D)
    c2 = jnp.sum(codebooks ** 2, axis=-1).reshape(NQ, 1, K)
    cb_flat = codebooks.reshape(NQ * K, D)
    sc_gather = _make_sc_gather(crows, D, nb, bsz)

    fb = rows // _NW
    fbsz = fb
    while fbsz > 128:
        fbsz //= 2
    sc_final = _make_sc_final(rows, D, fb // fbsz, fbsz)

    r = [None] * nch
    s = [None] * nch
    q_rows = [None] * nch
    idx_last = [None] * nch
    for q in range(_NQ):
        for ch in range(nch):
            if q == 0:
                idx = _tc_argmin_first(x2, codebooks, c2, q, ch, T, crows)
            elif q == _NQ - 1:
                s[ch], idx_last[ch] = _tc_sub_argmin_last(
                    r[ch], q_rows[ch], x2, codebooks, c2, q, ch, T)
                continue
            elif q == 1:
                r[ch], idx = _tc_sub_argmin(
                    x2, q_rows[ch], codebooks, c2, q, T,
                    roff=ch * (crows // T))
            else:
                r[ch], idx = _tc_sub_argmin(
                    r[ch], q_rows[ch], codebooks, c2, q, T)
            q_rows[ch] = sc_gather(cb_flat, idx)
    out = sc_final(cb_flat, s[0], s[1], idx_last[0], idx_last[1])
    return out.reshape(B, N, D)
